# baseline (speedup 1.0000x reference)
import jax
import jax.numpy as jnp
from jax.experimental import pallas as pl
from jax.experimental.pallas import tpu as pltpu

D_MODEL_ = 1024
D_OUT_ = 1024
E_ = 8
T_ = 4096
TMC_ = 256
NT_ = T_ // TMC_


def _moe_body(
    x_hbm,
    wg_ref,
    bg_ref,
    we_hbm,
    be_ref,
    out_hbm,
    x_vmem,
    w_buf,
    c_vmem,
    acc,
    xsem,
    wsem,
    osem,
):
    x_copies = [
        pltpu.make_async_copy(
            x_hbm.at[pl.ds(t * TMC_, TMC_), :],
            x_vmem.at[pl.ds(t * TMC_, TMC_), :],
            xsem.at[t],
        )
        for t in range(NT_)
    ]
    w_copies = [
        pltpu.make_async_copy(we_hbm.at[e], w_buf.at[e % 2], wsem.at[e % 2])
        for e in range(E_)
    ]
    out_copies = [
        pltpu.make_async_copy(
            acc.at[pl.ds(t * TMC_, TMC_), :],
            out_hbm.at[pl.ds(t * TMC_, TMC_), :],
            osem.at[t],
        )
        for t in range(NT_)
    ]

    x_copies[0].start()
    w_copies[0].start()
    w_copies[1].start()
    for t in range(1, NT_):
        x_copies[t].start()

    wg = wg_ref[...]
    bg = bg_ref[...]
    be = be_ref[...]

    w_copies[0].wait()
    for t in range(NT_):
        x_copies[t].wait()
        sl = pl.ds(t * TMC_, TMC_)
        x_t = x_vmem[sl, :]
        logits = jnp.dot(x_t, wg, preferred_element_type=jnp.float32) + bg
        m1 = jnp.max(logits, axis=-1, keepdims=True)
        oh1 = logits == m1
        l2 = jnp.where(oh1, -jnp.inf, logits)
        m2 = jnp.max(l2, axis=-1, keepdims=True)
        oh2 = l2 == m2
        w1 = jax.nn.sigmoid(m1 - m2)
        w2 = 1.0 - w1
        c = w1 * oh1.astype(jnp.float32) + w2 * oh2.astype(jnp.float32)
        c_vmem[sl, :] = c
        y = jnp.dot(x_t, w_buf[0], preferred_element_type=jnp.float32)
        acc[sl, :] = (
            jnp.dot(c, be, preferred_element_type=jnp.float32)
            + c[:, 0:1] * y
        )
    w_copies[2].start()

    for e in range(1, E_):
        w_copies[e].wait()
        for t in range(NT_):
            sl = pl.ds(t * TMC_, TMC_)
            x_t = x_vmem[sl, :]
            y = jnp.dot(x_t, w_buf[e % 2], preferred_element_type=jnp.float32)
            acc[sl, :] = acc[sl, :] + c_vmem[sl, e : e + 1] * y
            if e == E_ - 1:
                out_copies[t].start()
        if e + 2 < E_:
            w_copies[e + 2].start()

    for t in range(NT_):
        out_copies[t].wait()


def kernel(x, W_e, b_e, W_g, b_g):
    B, S, D = x.shape
    T = B * S
    xf = x.reshape(T, D)
    out = pl.pallas_call(
        _moe_body,
        in_specs=[
            pl.BlockSpec(memory_space=pl.ANY),
            pl.BlockSpec(memory_space=pltpu.VMEM),
            pl.BlockSpec(memory_space=pltpu.VMEM),
            pl.BlockSpec(memory_space=pl.ANY),
            pl.BlockSpec(memory_space=pltpu.VMEM),
        ],
        out_specs=pl.BlockSpec(memory_space=pl.ANY),
        out_shape=jax.ShapeDtypeStruct((T, D_OUT_), jnp.float32),
        scratch_shapes=[
            pltpu.VMEM((T_, D_MODEL_), jnp.float32),
            pltpu.VMEM((2, D_MODEL_, D_OUT_), jnp.float32),
            pltpu.VMEM((T_, E_), jnp.float32),
            pltpu.VMEM((T_, D_OUT_), jnp.float32),
            pltpu.SemaphoreType.DMA((NT_,)),
            pltpu.SemaphoreType.DMA((2,)),
            pltpu.SemaphoreType.DMA((NT_,)),
        ],
    )(xf, W_g, b_g.reshape(1, E_), W_e, b_e)
    return out.reshape(B, S, D_OUT_)
